# baseline (speedup 1.0000x reference)
import functools
import jax
import jax.numpy as jnp
from jax import lax
from jax.experimental import pallas as pl
from jax.experimental.pallas import tpu as pltpu
from jax.experimental.pallas import tpu_sc as plsc

_B, _N, _C = 8, 2048, 64
_CLAMP = 60.0
_RATIO = 0.8
_BLK = 512
_NB = _N // _BLK
_SL = _N // 4
_NCH = _N // 16
_SCH = _SL // 16

_mesh = plsc.VectorSubcoreMesh(core_axis_name="c", subcore_axis_name="s")


@functools.partial(
    pl.kernel,
    out_type=[jax.ShapeDtypeStruct((_B, _N), jnp.float32)] * 3,
    mesh=_mesh,
    compiler_params=pltpu.CompilerParams(needs_layout_passes=False),
    scratch_types=[
        pltpu.VMEM((_SL * _C,), jnp.float32),
        pltpu.VMEM((_C,), jnp.float32),
        pltpu.VMEM((_SL,), jnp.float32),
        pltpu.VMEM((16,), jnp.float32),
        pltpu.VMEM((16,), jnp.int32),
        pltpu.VMEM((_SL,), jnp.float32),
        pltpu.VMEM((_N,), jnp.float32),
        pltpu.VMEM((_N,), jnp.int32),
        pltpu.VMEM((_SL,), jnp.float32),
        pltpu.VMEM((_SL,), jnp.float32),
        pltpu.VMEM((_SL,), jnp.float32),
        pltpu.VMEM_SHARED((4, _N), jnp.float32),
    ],
)
def _sc_stage1(x_hbm, w_hbm, mask_hbm, nnf_hbm, nk_hbm, alpha_hbm,
               nmask_hbm, s_hbm, xv, wv, maskv, nnfv, nkv, ev, ebuf, bitsv,
               av, nmv, sv, e_sh):
    cid = lax.axis_index("c")
    sid = lax.axis_index("s")
    b = cid * 4 + sid // 4
    brow = sid // 4
    q = sid % 4
    base = q * _SL

    pltpu.sync_copy(x_hbm.at[b, pl.ds(base * _C, _SL * _C)], xv)
    pltpu.sync_copy(w_hbm.at[0], wv)
    pltpu.sync_copy(mask_hbm.at[b, pl.ds(base, _SL)], maskv)
    pltpu.sync_copy(nnf_hbm, nnfv)
    pltpu.sync_copy(nk_hbm, nkv)

    iota = lax.iota(jnp.int32, 16)
    iota64 = iota * _C
    wch = [wv[pl.ds(k * 16, 16)] for k in range(_C // 16)]
    wsc = [wch[c // 16][c % 16] for c in range(_C)]

    def groupA(g, carry):
        gbase = g * (16 * _C)
        acc = jnp.zeros((16,), jnp.float32)
        for c in range(_C):
            acc = acc + plsc.load_gather(xv, [iota64 + (gbase + c)]) * wsc[c]
        ap = jnp.minimum(jnp.maximum(acc, -_CLAMP), _CLAMP)
        e16 = jnp.exp(ap) * maskv[pl.ds(g * 16, 16)]
        ev[pl.ds(g * 16, 16)] = e16
        return carry

    lax.fori_loop(0, _SCH, groupA, 0)
    pltpu.sync_copy(ev, e_sh.at[brow, pl.ds(base, _SL)])
    plsc.subcore_barrier()

    pltpu.sync_copy(e_sh.at[brow], ebuf)
    total = jnp.zeros((16,), jnp.float32)
    for k in range(_NCH):
        total = total + ebuf[pl.ds(k * 16, 16)]
    inv16 = jnp.ones((16,), jnp.float32) / jnp.full((16,), jnp.sum(total)
                                                    + 1e-7, jnp.float32)
    for k in range(_NCH):
        bitsv[pl.ds(k * 16, 16)] = plsc.bitcast(
            ebuf[pl.ds(k * 16, 16)] * inv16, jnp.int32)

    bsel = iota == b
    nnf = jnp.sum(jnp.where(bsel, nnfv[pl.ds(0, 16)], 0.0))
    nkeep = jnp.sum(jnp.where(bsel, nkv[pl.ds(0, 16)], 0))

    def count_ge(v):
        acc = jnp.zeros((16,), jnp.int32)
        for k in range(_NCH):
            acc = acc + (bitsv[pl.ds(k * 16, 16)] >= v).astype(jnp.int32)
        return jnp.sum(acc)

    def tbody(_, lohi):
        lo, hi = lohi
        mid = lo + (hi - lo + 1) // 2
        ok = count_ge(mid) >= nkeep
        return jnp.where(ok, mid, lo), jnp.where(ok, hi, mid - 1)

    tbits, _ = lax.fori_loop(0, 30, tbody,
                             (jnp.int32(0), jnp.int32(0x3F800000)))
    n_gt = count_ge(tbits + 1)
    r = nkeep - n_gt

    def mbody(_, lohi):
        lo, hi = lohi
        mid = lo + (hi - lo + 1) // 2
        acc = jnp.zeros((16,), jnp.int32)
        for k in range(_NCH):
            idxk = iota + k * 16
            acc = acc + ((bitsv[pl.ds(k * 16, 16)] == tbits)
                         & (idxk <= mid)).astype(jnp.int32)
        ok = jnp.sum(acc) <= r
        return jnp.where(ok, mid, lo), jnp.where(ok, hi, mid - 1)

    mstar, _ = lax.fori_loop(0, 12, mbody,
                             (jnp.int32(-1), jnp.int32(_N - 1)))

    one = jnp.ones((16,), jnp.float32)
    zerof = jnp.zeros((16,), jnp.float32)
    for j in range(_SCH):
        off = base + j * 16
        bits16 = bitsv[pl.ds(off, 16)]
        a16 = plsc.bitcast(bits16, jnp.float32)
        keep = (bits16 > tbits) | ((bits16 == tbits) & ((iota + off) <= mstar))
        nm16 = jnp.where(keep & (maskv[pl.ds(j * 16, 16)] > 0.0), one, zerof)
        av[pl.ds(j * 16, 16)] = a16
        nmv[pl.ds(j * 16, 16)] = nm16
        sv[pl.ds(j * 16, 16)] = a16 * nnf * nm16
    pltpu.sync_copy(av, alpha_hbm.at[b, pl.ds(base, _SL)])
    pltpu.sync_copy(nmv, nmask_hbm.at[b, pl.ds(base, _SL)])
    pltpu.sync_copy(sv, s_hbm.at[b, pl.ds(base, _SL)])


def _stage2_body(A_ref, x_ref, rm_ref, cm_ref, s_ref, Ao_ref, xo_ref):
    rm = rm_ref[0, 0].reshape(_BLK, 1)
    cm = cm_ref[0]
    Ao_ref[0] = A_ref[0] * rm * cm
    xo_ref[0] = x_ref[0] * s_ref[0, 0].reshape(_BLK, 1)


_stage2 = pl.pallas_call(
    _stage2_body,
    grid=(_B, _NB),
    in_specs=[
        pl.BlockSpec((1, _BLK, _N), lambda b, j: (b, j, 0)),
        pl.BlockSpec((1, _BLK, _C), lambda b, j: (b, j, 0)),
        pl.BlockSpec((1, 1, 1, _BLK), lambda b, j: (b, j, 0, 0)),
        pl.BlockSpec((1, 1, _N), lambda b, j: (b, 0, 0)),
        pl.BlockSpec((1, 1, 1, _BLK), lambda b, j: (b, j, 0, 0)),
    ],
    out_specs=[
        pl.BlockSpec((1, _BLK, _N), lambda b, j: (b, j, 0)),
        pl.BlockSpec((1, _BLK, _C), lambda b, j: (b, j, 0)),
    ],
    out_shape=[
        jax.ShapeDtypeStruct((_B, _N, _N), jnp.float32),
        jax.ShapeDtypeStruct((_B, _N, _C), jnp.float32),
    ],
    compiler_params=pltpu.CompilerParams(
        dimension_semantics=("parallel", "arbitrary")),
)


def _round_bf16(v):
    u = lax.bitcast_convert_type(v, jnp.int32)
    u = (u + 0x7FFF + ((u >> 16) & 1)) & ~jnp.int32(0xFFFF)
    return lax.bitcast_convert_type(u, jnp.float32)


def kernel(x, A, mask, W, N_nodes):
    x_r = _round_bf16(x).reshape(_B, _N * _C)
    W_r = _round_bf16(W)
    nnf = N_nodes.astype(jnp.float32)
    nkeep = N_nodes - jnp.round(nnf * (1.0 - _RATIO)).astype(jnp.int32)
    nnf16 = jnp.pad(nnf, (0, 16 - _B))
    nk16 = jnp.pad(nkeep, (0, 16 - _B))
    alpha, nm, s = _sc_stage1(x_r, W_r, mask, nnf16, nk16)
    nm4 = nm.reshape(_B, _NB, 1, _BLK)
    s4 = s.reshape(_B, _NB, 1, _BLK)
    Ao, xo = _stage2(A, x, nm4, nm.reshape(_B, 1, _N), s4)
    return xo, Ao, nm, alpha


# speedup vs baseline: 1.0003x; 1.0003x over previous
import functools
import jax
import jax.numpy as jnp
from jax import lax
from jax.experimental import pallas as pl
from jax.experimental.pallas import tpu as pltpu
from jax.experimental.pallas import tpu_sc as plsc

_B, _N, _C = 8, 2048, 64
_CLAMP = 60.0
_RATIO = 0.8
_BLK = 512
_NB = _N // _BLK
_SL = _N // 4
_NCH = _N // 16
_SCH = _SL // 16

_mesh = plsc.VectorSubcoreMesh(core_axis_name="c", subcore_axis_name="s")


@functools.partial(
    pl.kernel,
    out_type=[jax.ShapeDtypeStruct((_B, _N), jnp.float32)] * 3,
    mesh=_mesh,
    compiler_params=pltpu.CompilerParams(needs_layout_passes=False),
    scratch_types=[
        pltpu.VMEM((_SL * _C,), jnp.float32),
        pltpu.VMEM((_C,), jnp.float32),
        pltpu.VMEM((_SL,), jnp.float32),
        pltpu.VMEM((16,), jnp.float32),
        pltpu.VMEM((16,), jnp.int32),
        pltpu.VMEM((_SL,), jnp.float32),
        pltpu.VMEM((_N,), jnp.float32),
        pltpu.VMEM((_N,), jnp.int32),
        pltpu.VMEM((_SL,), jnp.float32),
        pltpu.VMEM((_SL,), jnp.float32),
        pltpu.VMEM((_SL,), jnp.float32),
        pltpu.VMEM_SHARED((4, _N), jnp.float32),
    ],
)
def _sc_stage1(x_hbm, w_hbm, mask_hbm, nnf_hbm, nk_hbm, alpha_hbm,
               nmask_hbm, s_hbm, xv, wv, maskv, nnfv, nkv, ev, ebuf, bitsv,
               av, nmv, sv, e_sh):
    cid = lax.axis_index("c")
    sid = lax.axis_index("s")
    b = cid * 4 + sid // 4
    brow = sid // 4
    q = sid % 4
    base = q * _SL

    pltpu.sync_copy(x_hbm.at[b, pl.ds(base * _C, _SL * _C)], xv)
    pltpu.sync_copy(w_hbm.at[0], wv)
    pltpu.sync_copy(mask_hbm.at[b, pl.ds(base, _SL)], maskv)
    pltpu.sync_copy(nnf_hbm, nnfv)
    pltpu.sync_copy(nk_hbm, nkv)

    iota = lax.iota(jnp.int32, 16)
    iota64 = iota * _C
    wch = [wv[pl.ds(k * 16, 16)] for k in range(_C // 16)]
    wsc = [wch[c // 16][c % 16] for c in range(_C)]

    def groupA(g, carry):
        gbase = g * (16 * _C)
        acc = jnp.zeros((16,), jnp.float32)
        for c in range(_C):
            acc = acc + plsc.load_gather(xv, [iota64 + (gbase + c)]) * wsc[c]
        ap = jnp.minimum(jnp.maximum(acc, -_CLAMP), _CLAMP)
        e16 = jnp.exp(ap) * maskv[pl.ds(g * 16, 16)]
        ev[pl.ds(g * 16, 16)] = e16
        return carry

    lax.fori_loop(0, _SCH, groupA, 0)
    pltpu.sync_copy(ev, e_sh.at[brow, pl.ds(base, _SL)])
    plsc.subcore_barrier()

    pltpu.sync_copy(e_sh.at[brow], ebuf)
    tot4 = [jnp.zeros((16,), jnp.float32) for _ in range(4)]
    for k in range(_NCH):
        tot4[k % 4] = tot4[k % 4] + ebuf[pl.ds(k * 16, 16)]
    total = (tot4[0] + tot4[1]) + (tot4[2] + tot4[3])
    inv16 = jnp.ones((16,), jnp.float32) / jnp.full((16,), jnp.sum(total)
                                                    + 1e-7, jnp.float32)
    for k in range(_NCH):
        bitsv[pl.ds(k * 16, 16)] = plsc.bitcast(
            ebuf[pl.ds(k * 16, 16)] * inv16, jnp.int32)

    bsel = iota == b
    nnf = jnp.sum(jnp.where(bsel, nnfv[pl.ds(0, 16)], 0.0))
    nkeep = jnp.sum(jnp.where(bsel, nkv[pl.ds(0, 16)], 0))

    def count_ge(v):
        accs = [jnp.zeros((16,), jnp.int32) for _ in range(4)]
        for k in range(_NCH):
            accs[k % 4] = accs[k % 4] + (
                bitsv[pl.ds(k * 16, 16)] >= v).astype(jnp.int32)
        return jnp.sum((accs[0] + accs[1]) + (accs[2] + accs[3]))

    def tbody(_, lohi):
        lo, hi = lohi
        mid = lo + (hi - lo + 1) // 2
        ok = count_ge(mid) >= nkeep
        return jnp.where(ok, mid, lo), jnp.where(ok, hi, mid - 1)

    tbits, _ = lax.fori_loop(0, 30, tbody,
                             (jnp.int32(0), jnp.int32(0x3F800000)))
    n_gt = count_ge(tbits + 1)
    r = nkeep - n_gt

    def mbody(_, lohi):
        lo, hi = lohi
        mid = lo + (hi - lo + 1) // 2
        accs = [jnp.zeros((16,), jnp.int32) for _ in range(4)]
        for k in range(_NCH):
            idxk = iota + k * 16
            accs[k % 4] = accs[k % 4] + (
                (bitsv[pl.ds(k * 16, 16)] == tbits)
                & (idxk <= mid)).astype(jnp.int32)
        ok = jnp.sum((accs[0] + accs[1]) + (accs[2] + accs[3])) <= r
        return jnp.where(ok, mid, lo), jnp.where(ok, hi, mid - 1)

    mstar, _ = lax.fori_loop(0, 12, mbody,
                             (jnp.int32(-1), jnp.int32(_N - 1)))

    one = jnp.ones((16,), jnp.float32)
    zerof = jnp.zeros((16,), jnp.float32)
    for j in range(_SCH):
        off = base + j * 16
        bits16 = bitsv[pl.ds(off, 16)]
        a16 = plsc.bitcast(bits16, jnp.float32)
        keep = (bits16 > tbits) | ((bits16 == tbits) & ((iota + off) <= mstar))
        nm16 = jnp.where(keep & (maskv[pl.ds(j * 16, 16)] > 0.0), one, zerof)
        av[pl.ds(j * 16, 16)] = a16
        nmv[pl.ds(j * 16, 16)] = nm16
        sv[pl.ds(j * 16, 16)] = a16 * nnf * nm16
    pltpu.sync_copy(av, alpha_hbm.at[b, pl.ds(base, _SL)])
    pltpu.sync_copy(nmv, nmask_hbm.at[b, pl.ds(base, _SL)])
    pltpu.sync_copy(sv, s_hbm.at[b, pl.ds(base, _SL)])


def _stage2_body(A_ref, x_ref, rm_ref, cm_ref, s_ref, Ao_ref, xo_ref):
    rm = rm_ref[0, 0].reshape(_BLK, 1)
    cm = cm_ref[0]
    Ao_ref[0] = A_ref[0] * rm * cm
    xo_ref[0] = x_ref[0] * s_ref[0, 0].reshape(_BLK, 1)


_stage2 = pl.pallas_call(
    _stage2_body,
    grid=(_B, _NB),
    in_specs=[
        pl.BlockSpec((1, _BLK, _N), lambda b, j: (b, j, 0)),
        pl.BlockSpec((1, _BLK, _C), lambda b, j: (b, j, 0)),
        pl.BlockSpec((1, 1, 1, _BLK), lambda b, j: (b, j, 0, 0)),
        pl.BlockSpec((1, 1, _N), lambda b, j: (b, 0, 0)),
        pl.BlockSpec((1, 1, 1, _BLK), lambda b, j: (b, j, 0, 0)),
    ],
    out_specs=[
        pl.BlockSpec((1, _BLK, _N), lambda b, j: (b, j, 0)),
        pl.BlockSpec((1, _BLK, _C), lambda b, j: (b, j, 0)),
    ],
    out_shape=[
        jax.ShapeDtypeStruct((_B, _N, _N), jnp.float32),
        jax.ShapeDtypeStruct((_B, _N, _C), jnp.float32),
    ],
    compiler_params=pltpu.CompilerParams(
        dimension_semantics=("parallel", "arbitrary")),
)


def _round_bf16(v):
    u = lax.bitcast_convert_type(v, jnp.int32)
    u = (u + 0x7FFF + ((u >> 16) & 1)) & ~jnp.int32(0xFFFF)
    return lax.bitcast_convert_type(u, jnp.float32)


def kernel(x, A, mask, W, N_nodes):
    x_r = _round_bf16(x).reshape(_B, _N * _C)
    W_r = _round_bf16(W)
    nnf = N_nodes.astype(jnp.float32)
    nkeep = N_nodes - jnp.round(nnf * (1.0 - _RATIO)).astype(jnp.int32)
    nnf16 = jnp.pad(nnf, (0, 16 - _B))
    nk16 = jnp.pad(nkeep, (0, 16 - _B))
    alpha, nm, s = _sc_stage1(x_r, W_r, mask, nnf16, nk16)
    nm4 = nm.reshape(_B, _NB, 1, _BLK)
    s4 = s.reshape(_B, _NB, 1, _BLK)
    Ao, xo = _stage2(A, x, nm4, nm.reshape(_B, 1, _N), s4)
    return xo, Ao, nm, alpha
